# Initial kernel scaffold; baseline (speedup 1.0000x reference)
import jax
import jax.numpy as jnp
from jax.experimental import pallas as pl


def kernel(q, k, v):
    raise NotImplementedError("write your pallas kernel here")


# speedup vs baseline: 3.8452x; 3.8452x over previous
import functools
import math

import jax
import jax.numpy as jnp
import numpy as np
from jax.experimental import pallas as pl
from jax.experimental.pallas import tpu as pltpu

_L = 2048
_D = 128
_H = 16
_NTOP = 38
_NPAD = 40
_QCHUNK = 512


def _build_sample_weights() -> np.ndarray:
    try:
        cpu = jax.devices("cpu")[0]
        with jax.default_device(cpu):
            idxs = np.asarray(
                jax.random.randint(jax.random.key(42), (_L, _NTOP), 0, _L))
    except Exception:
        idxs = np.asarray(
            jax.random.randint(jax.random.key(42), (_L, _NTOP), 0, _L))
    counts = np.zeros((_L, _L), np.float32)
    np.add.at(counts, (np.arange(_L)[:, None], idxs), 1.0)
    return counts / np.float32(_NTOP)


_W_NP = _build_sample_weights()


def _score_topk_body(q_ref, k_ref, w_ref, idx_ref):
    kh = k_ref[0]
    m_parts = []
    for c in range(_L // _QCHUNK):
        qc = q_ref[0, c * _QCHUNK:(c + 1) * _QCHUNK, :]
        s = jax.lax.dot_general(
            qc, kh, (((1,), (1,)), ((), ())),
            preferred_element_type=jnp.float32)
        wc = w_ref[c * _QCHUNK:(c + 1) * _QCHUNK, :]
        mx = jnp.max(jnp.where(wc > 0.0, s, -1e30), axis=1, keepdims=True)
        mn = jnp.sum(s * wc, axis=1, keepdims=True)
        m_parts.append(mx - mn)
    m2d = jnp.concatenate(m_parts, axis=0).reshape(16, 128)

    flat = (jax.lax.broadcasted_iota(jnp.int32, (16, 128), 0) * 128
            + jax.lax.broadcasted_iota(jnp.int32, (16, 128), 1))
    slot = jax.lax.broadcasted_iota(jnp.int32, (1, _NPAD), 1)

    def body(i, carry):
        m, acc = carry
        cur = jnp.max(m)
        sel = jnp.min(jnp.where(m == cur, flat, jnp.int32(2 * _L)))
        m = jnp.where(flat == sel, -1e30, m)
        acc = jnp.where(slot == i, sel, acc)
        return m, acc

    _, acc = jax.lax.fori_loop(
        0, _NTOP, body, (m2d, jnp.zeros((1, _NPAD), jnp.int32)))
    idx_ref[0] = acc


def _attn_scatter_body(idx_sref, q_ref, k_ref, v_ref, out_ref):
    h = pl.program_id(0)
    kh = k_ref[0]
    vh = v_ref[0]
    rows = [q_ref[0, pl.ds(idx_sref[h, i], 1), :] for i in range(_NTOP)]
    qred = jnp.concatenate(rows, axis=0)
    s = jax.lax.dot_general(
        qred, kh, (((1,), (1,)), ((), ())),
        preferred_element_type=jnp.float32) * (1.0 / math.sqrt(_D))
    mx = jnp.max(s, axis=1, keepdims=True)
    e = jnp.exp(s - mx)
    p = e / jnp.sum(e, axis=1, keepdims=True)
    ctx = jax.lax.dot_general(
        p, vh, (((1,), (0,)), ((), ())),
        preferred_element_type=jnp.float32)
    vmean = jnp.mean(vh, axis=0, keepdims=True)
    out_ref[0] = jnp.broadcast_to(vmean, (_L, _D))
    for i in range(_NTOP):
        out_ref[0, pl.ds(idx_sref[h, i], 1), :] = ctx[i:i + 1, :]


@jax.jit
def kernel(q, k, v):
    b, h, l, d = q.shape
    q3, k3, v3 = q[0], k[0], v[0]
    w = jnp.asarray(_W_NP)

    idx = pl.pallas_call(
        _score_topk_body,
        grid=(_H,),
        in_specs=[
            pl.BlockSpec((1, _L, _D), lambda hh: (hh, 0, 0)),
            pl.BlockSpec((1, _L, _D), lambda hh: (hh, 0, 0)),
            pl.BlockSpec((_L, _L), lambda hh: (0, 0)),
        ],
        out_specs=pl.BlockSpec((1, 1, _NPAD), lambda hh: (hh, 0, 0)),
        out_shape=jax.ShapeDtypeStruct((_H, 1, _NPAD), jnp.int32),
    )(q3, k3, w)

    grid_spec = pltpu.PrefetchScalarGridSpec(
        num_scalar_prefetch=1,
        grid=(_H,),
        in_specs=[
            pl.BlockSpec((1, _L, _D), lambda hh, idx_s: (hh, 0, 0)),
            pl.BlockSpec((1, _L, _D), lambda hh, idx_s: (hh, 0, 0)),
            pl.BlockSpec((1, _L, _D), lambda hh, idx_s: (hh, 0, 0)),
        ],
        out_specs=pl.BlockSpec((1, _L, _D), lambda hh, idx_s: (hh, 0, 0)),
    )
    out = pl.pallas_call(
        _attn_scatter_body,
        grid_spec=grid_spec,
        out_shape=jax.ShapeDtypeStruct((_H, _L, _D), jnp.float32),
    )(idx.reshape(_H, _NPAD), q3, k3, v3)
    return out[None]
